# Initial kernel scaffold; baseline (speedup 1.0000x reference)
import jax
import jax.numpy as jnp
from jax.experimental import pallas as pl


def kernel(x, edge_index, W1, b1, W2, b2, W3, b3):
    raise NotImplementedError("write your pallas kernel here")


# speedup vs baseline: 18.2630x; 18.2630x over previous
import jax
import jax.numpy as jnp
from jax import lax
from jax.experimental import pallas as pl
from jax.experimental.pallas import tpu as pltpu
from jax.experimental.pallas import tpu_sc as plsc

N = 10000
NR = 10240
NSUB = 16
NCORE = 2
NW = NCORE * NSUB
RPS = NR // NSUB
CB = 128
NCH_W = 158
NCH_N = 80
EPAD_W = NSUB * NCH_W * CB
EPAD_N = NW * NCH_N * CB
BR = 1000


def _mesh():
  return plsc.VectorSubcoreMesh(core_axis_name="c", subcore_axis_name="s")


DW = 8


def _deg_body(dstp, zeros, ones, out, acc, dst_v, ones_v, wb):
  c = lax.axis_index("c")
  s = lax.axis_index("s")
  w = c * NSUB + s
  base = s * RPS
  pltpu.sync_copy(zeros.at[pl.ds(base, RPS)], wb)
  pltpu.sync_copy(wb, acc.at[pl.ds(base, RPS)])
  pltpu.sync_copy(ones, ones_v)
  pltpu.sync_copy(dstp.at[w], dst_v)
  plsc.subcore_barrier()

  def body(j, carry):
    pltpu.sync_copy(ones_v, acc.at[dst_v.at[j]], add=True)
    return carry

  lax.fori_loop(0, NCH_N, body, 0)
  plsc.subcore_barrier()
  pltpu.sync_copy(acc.at[pl.ds(base, RPS)], wb)
  pltpu.sync_copy(wb, out.at[c, pl.ds(base, RPS)])


def _agg_body(table, srcp, dstp, zeros, out,
              acc, src_v, dst_v, rows_a, rows_b, wb, sem_a, sem_b):
  c = lax.axis_index("c")
  s = lax.axis_index("s")
  base = s * RPS
  half = RPS // 2
  pltpu.sync_copy(zeros.at[pl.ds(base, half)], wb)
  pltpu.sync_copy(wb, acc.at[pl.ds(base, half)])
  pltpu.sync_copy(wb, acc.at[pl.ds(base + half, half)])
  pltpu.sync_copy(srcp.at[s], src_v)
  pltpu.sync_copy(dstp.at[s], dst_v)
  plsc.subcore_barrier()

  tab = table.at[c]

  pltpu.async_copy(tab.at[src_v.at[0]], rows_a, sem_a)

  def body(i, carry):
    ja = 2 * i
    pltpu.make_async_copy(tab.at[src_v.at[ja]], rows_a, sem_a).wait()
    pltpu.async_copy(tab.at[src_v.at[ja + 1]], rows_b, sem_b)
    pltpu.sync_copy(rows_a, acc.at[dst_v.at[ja]], add=True)
    pltpu.make_async_copy(tab.at[src_v.at[ja + 1]], rows_b, sem_b).wait()
    jn = jnp.minimum(ja + 2, NCH_W - 1)
    pltpu.async_copy(tab.at[src_v.at[jn]], rows_a, sem_a)
    pltpu.sync_copy(rows_b, acc.at[dst_v.at[ja + 1]], add=True)
    return carry

  lax.fori_loop(0, NCH_W // 2, body, 0)
  pltpu.make_async_copy(tab.at[src_v.at[NCH_W - 1]], rows_a, sem_a).wait()

  plsc.subcore_barrier()
  pltpu.sync_copy(acc.at[pl.ds(base, half)], wb)
  pltpu.sync_copy(wb, out.at[c, pl.ds(base, half)])
  pltpu.sync_copy(acc.at[pl.ds(base + half, half)], wb)
  pltpu.sync_copy(wb, out.at[c, pl.ds(base + half, half)])


def _make_deg():
  return pl.kernel(
      _deg_body,
      out_type=jax.ShapeDtypeStruct((NCORE, NR, DW), jnp.float32),
      mesh=_mesh(),
      compiler_params=pltpu.CompilerParams(use_tc_tiling_on_sc=False),
      scratch_types=[
          pltpu.MemorySpace.VMEM_SHARED((NR, DW), jnp.float32),
          pltpu.VMEM((NCH_N, CB), jnp.int32),
          pltpu.VMEM((CB, DW), jnp.float32),
          pltpu.VMEM((RPS, DW), jnp.float32),
      ],
  )


def _make_agg():
  return pl.kernel(
      _agg_body,
      out_type=jax.ShapeDtypeStruct((NCORE, NR, 64), jnp.float32),
      mesh=_mesh(),
      compiler_params=pltpu.CompilerParams(use_tc_tiling_on_sc=False),
      scratch_types=[
          pltpu.MemorySpace.VMEM_SHARED((NR, 64), jnp.float32),
          pltpu.VMEM((NCH_W, CB), jnp.int32),
          pltpu.VMEM((NCH_W, CB), jnp.int32),
          pltpu.VMEM((CB, 64), jnp.float32),
          pltpu.VMEM((CB, 64), jnp.float32),
          pltpu.VMEM((RPS // 2, 64), jnp.float32),
          pltpu.SemaphoreType.DMA,
          pltpu.SemaphoreType.DMA,
      ],
  )


def _split(h):
  return jnp.stack([h[:, :64], h[:, 64:]], axis=0)


def _cat(ref_a, ref_b):
  return jnp.concatenate([ref_a[0] + ref_b[0], ref_a[1] + ref_b[1]], axis=1)


def _b3spec(i):
  return (0, i, 0)


def _tc_pre_body(degp_ref, x_ref, w_ref, dinv_ref, hp_ref):
  deg = degp_ref[0] + degp_ref[1] + 1.0
  dinv = jnp.where(deg > 0, lax.rsqrt(deg), 0.0)
  dinv_ref[...] = dinv
  h = jnp.dot(x_ref[...], w_ref[...], preferred_element_type=jnp.float32)
  hp_ref[...] = _split(h * dinv)


def _tc_pre(degp, x, w):
  return pl.pallas_call(
      _tc_pre_body,
      grid=(N // BR,),
      in_specs=[
          pl.BlockSpec((NCORE, BR, 1), _b3spec),
          pl.BlockSpec((BR, x.shape[1]), lambda i: (i, 0)),
          pl.BlockSpec(w.shape, lambda i: (0, 0)),
      ],
      out_specs=[
          pl.BlockSpec((BR, 1), lambda i: (i, 0)),
          pl.BlockSpec((NCORE, BR, 64), _b3spec),
      ],
      out_shape=[
          jax.ShapeDtypeStruct((N, 1), jnp.float32),
          jax.ShapeDtypeStruct((NCORE, N, 64), jnp.float32),
      ],
  )(degp, x, w)


def _tc_mid_body(agg_ref, hp_ref, dinv_ref, b_ref, w_ref, out_ref):
  dinv = dinv_ref[...]
  t = _cat(agg_ref, hp_ref) * dinv + b_ref[...]
  t = jnp.maximum(t, 0.0)
  o = jnp.dot(t, w_ref[...], preferred_element_type=jnp.float32) * dinv
  out_ref[...] = _split(o)


def _tc_mid(agg, hp, dinv, b, w):
  return pl.pallas_call(
      _tc_mid_body,
      grid=(N // BR,),
      in_specs=[
          pl.BlockSpec((NCORE, BR, 64), _b3spec),
          pl.BlockSpec((NCORE, BR, 64), _b3spec),
          pl.BlockSpec((BR, 1), lambda i: (i, 0)),
          pl.BlockSpec((1, 128), lambda i: (0, 0)),
          pl.BlockSpec((128, 128), lambda i: (0, 0)),
      ],
      out_specs=pl.BlockSpec((NCORE, BR, 64), _b3spec),
      out_shape=jax.ShapeDtypeStruct((NCORE, N, 64), jnp.float32),
  )(agg, hp, dinv, b, w)


def _tc_act_body(agg_ref, hp_ref, dinv_ref, b_ref, out_ref):
  dinv = dinv_ref[...]
  t = _cat(agg_ref, hp_ref) * dinv + b_ref[...]
  t = jnp.maximum(t, 0.0) * dinv
  out_ref[...] = _split(t)


def _tc_act(agg, hp, dinv, b):
  return pl.pallas_call(
      _tc_act_body,
      grid=(N // BR,),
      in_specs=[
          pl.BlockSpec((NCORE, BR, 64), _b3spec),
          pl.BlockSpec((NCORE, BR, 64), _b3spec),
          pl.BlockSpec((BR, 1), lambda i: (i, 0)),
          pl.BlockSpec((1, 128), lambda i: (0, 0)),
      ],
      out_specs=pl.BlockSpec((NCORE, BR, 64), _b3spec),
      out_shape=jax.ShapeDtypeStruct((NCORE, N, 64), jnp.float32),
  )(agg, hp, dinv, b)


def _tc_fin_body(agg_ref, hp_ref, dinv_ref, w_ref, b_ref, out_ref):
  t = _cat(agg_ref, hp_ref) * dinv_ref[...]
  o = jnp.dot(t, w_ref[...], preferred_element_type=jnp.float32) + b_ref[...]
  out_ref[...] = jax.nn.sigmoid(o)


def _tc_fin(agg, hp, dinv, w, b):
  return pl.pallas_call(
      _tc_fin_body,
      grid=(N // BR,),
      in_specs=[
          pl.BlockSpec((NCORE, BR, 64), _b3spec),
          pl.BlockSpec((NCORE, BR, 64), _b3spec),
          pl.BlockSpec((BR, 1), lambda i: (i, 0)),
          pl.BlockSpec((128, 1), lambda i: (0, 0)),
          pl.BlockSpec((1, 1), lambda i: (0, 0)),
      ],
      out_specs=pl.BlockSpec((BR, 1), lambda i: (i, 0)),
      out_shape=jax.ShapeDtypeStruct((N, 1), jnp.float32),
  )(agg, hp, dinv, w, b)


def _pad_edges(src, dst, epad, rows):
  pad_i = jnp.arange(epad - src.shape[0], dtype=jnp.int32)
  srcp = jnp.concatenate([src, pad_i % N]).reshape(rows, -1, CB)
  dstp = jnp.concatenate([dst, N + pad_i % (NR - N)]).reshape(rows, -1, CB)
  return srcp, dstp


@jax.jit
def kernel(x, edge_index, W1, b1, W2, b2, W3, b3):
  ei = edge_index.astype(jnp.int32)
  src, dst = ei[0], ei[1]
  srcp16, dstp16 = _pad_edges(src, dst, EPAD_W, NSUB)
  _, dstp32 = _pad_edges(src, dst, EPAD_N, NW)

  zeros8 = jnp.zeros((NR, DW), jnp.float32)
  zeros64 = jnp.zeros((NR, 64), jnp.float32)
  ones = jnp.ones((CB, DW), jnp.float32)

  agg = _make_agg()

  degp = _make_deg()(dstp32, zeros8, ones)
  dinv, h1p = _tc_pre(degp[:, :N, :1], x, W1)
  a1 = agg(h1p, srcp16, dstp16, zeros64)
  h2p = _tc_mid(a1[:, :N], h1p, dinv, b1.reshape(1, -1), W2)
  a2 = agg(h2p, srcp16, dstp16, zeros64)
  h3p = _tc_act(a2[:, :N], h2p, dinv, b2.reshape(1, -1))
  a3 = agg(h3p, srcp16, dstp16, zeros64)
  return _tc_fin(a3[:, :N], h3p, dinv, W3, b3.reshape(1, 1))


# speedup vs baseline: 23.1916x; 1.2699x over previous
import jax
import jax.numpy as jnp
from jax import lax
from jax.experimental import pallas as pl
from jax.experimental.pallas import tpu as pltpu
from jax.experimental.pallas import tpu_sc as plsc

N = 10000
NR = 10240
NSUB = 16
NCORE = 2
NW = NCORE * NSUB
RPS = NR // NSUB
CB = 128
NCH_W = 160
NCH_N = 80
EPAD = NSUB * NCH_W * CB
BR = 1000


def _mesh():
  return plsc.VectorSubcoreMesh(core_axis_name="c", subcore_axis_name="s")


DW = 8


def _deg_body(dstp, zeros, ones, out, acc, dst_v, ones_v, wb):
  c = lax.axis_index("c")
  s = lax.axis_index("s")
  w = c * NSUB + s
  base = s * RPS
  pltpu.sync_copy(zeros.at[pl.ds(base, RPS)], wb)
  pltpu.sync_copy(wb, acc.at[pl.ds(base, RPS)])
  pltpu.sync_copy(ones, ones_v)
  pltpu.sync_copy(dstp.at[w], dst_v)
  plsc.subcore_barrier()

  def body(j, carry):
    pltpu.sync_copy(ones_v, acc.at[dst_v.at[j]], add=True)
    return carry

  lax.fori_loop(0, NCH_N, body, 0)
  plsc.subcore_barrier()
  pltpu.sync_copy(acc.at[pl.ds(base, RPS)], wb)
  pltpu.sync_copy(wb, out.at[c, pl.ds(base, RPS)])


def _agg_body(table, srcp, dstp, zeros, out,
              acc, src_v, dst_v, r0, r1, r2, r3, wb,
              g0, g1, g2, g3, s0, s1, s2, s3):
  c = lax.axis_index("c")
  s = lax.axis_index("s")
  base = s * RPS
  rows = (r0, r1, r2, r3)
  gsem = (g0, g1, g2, g3)
  ssem = (s0, s1, s2, s3)

  pltpu.sync_copy(zeros.at[pl.ds(base, CB)], wb)
  for t in range(RPS // CB):
    pltpu.sync_copy(wb, acc.at[pl.ds(base + t * CB, CB)])
  pltpu.sync_copy(srcp.at[s], src_v)
  pltpu.sync_copy(dstp.at[s], dst_v)
  plsc.subcore_barrier()

  tab = table.at[c]

  def g_start(k, m):
    pltpu.async_copy(tab.at[src_v.at[m]], rows[k], gsem[k])

  def g_wait(k, m):
    pltpu.make_async_copy(tab.at[src_v.at[m]], rows[k], gsem[k]).wait()

  def s_start(k, m):
    pltpu.async_copy(rows[k], acc.at[dst_v.at[m]], ssem[k], add=True)

  def s_wait(k, m):
    pltpu.make_async_copy(rows[k], acc.at[dst_v.at[m]], ssem[k]).wait()

  def turn(m, kk, first_group, last_group):
    k2 = (kk + 2) % 4
    g_wait(kk, m)
    s_start(kk, m)
    if not (first_group and kk < 2):
      s_wait(k2, m - 2)
    if not (last_group and kk >= 2):
      g_start(k2, m + 2)

  g_start(0, 0)
  g_start(1, 1)
  for kk in range(4):
    turn(kk, kk, True, False)

  def body(i, carry):
    for kk in range(4):
      turn(4 * i + kk, kk, False, False)
    return carry

  lax.fori_loop(1, NCH_W // 4 - 1, body, 0)

  for kk in range(4):
    turn(NCH_W - 4 + kk, kk, False, True)
  s_wait(2, NCH_W - 2)
  s_wait(3, NCH_W - 1)

  plsc.subcore_barrier()
  for t in range(RPS // CB):
    sl = pl.ds(base + t * CB, CB)
    pltpu.sync_copy(acc.at[sl], wb)
    pltpu.sync_copy(wb, out.at[c, sl])


def _make_deg():
  return pl.kernel(
      _deg_body,
      out_type=jax.ShapeDtypeStruct((NCORE, NR, DW), jnp.float32),
      mesh=_mesh(),
      compiler_params=pltpu.CompilerParams(use_tc_tiling_on_sc=False),
      scratch_types=[
          pltpu.MemorySpace.VMEM_SHARED((NR, DW), jnp.float32),
          pltpu.VMEM((NCH_N, CB), jnp.int32),
          pltpu.VMEM((CB, DW), jnp.float32),
          pltpu.VMEM((RPS, DW), jnp.float32),
      ],
  )


def _make_agg():
  return pl.kernel(
      _agg_body,
      out_type=jax.ShapeDtypeStruct((NCORE, NR, 64), jnp.float32),
      mesh=_mesh(),
      compiler_params=pltpu.CompilerParams(use_tc_tiling_on_sc=False),
      scratch_types=[
          pltpu.MemorySpace.VMEM_SHARED((NR, 64), jnp.float32),
          pltpu.VMEM((NCH_W, CB), jnp.int32),
          pltpu.VMEM((NCH_W, CB), jnp.int32),
          pltpu.VMEM((CB, 64), jnp.float32),
          pltpu.VMEM((CB, 64), jnp.float32),
          pltpu.VMEM((CB, 64), jnp.float32),
          pltpu.VMEM((CB, 64), jnp.float32),
          pltpu.VMEM((CB, 64), jnp.float32),
      ] + [pltpu.SemaphoreType.DMA] * 8,
  )


def _split(h):
  return jnp.stack([h[:, :64], h[:, 64:]], axis=0)


def _cat(ref_a, ref_b):
  return jnp.concatenate([ref_a[0] + ref_b[0], ref_a[1] + ref_b[1]], axis=1)


def _b3spec(i):
  return (0, i, 0)


def _tc_pre_body(degp_ref, x_ref, w_ref, dinv_ref, hp_ref):
  deg = degp_ref[0] + degp_ref[1] + 1.0
  dinv = jnp.where(deg > 0, lax.rsqrt(deg), 0.0)
  dinv_ref[...] = dinv
  h = jnp.dot(x_ref[...], w_ref[...], preferred_element_type=jnp.float32)
  hp_ref[...] = _split(h * dinv)


def _tc_pre(degp, x, w):
  return pl.pallas_call(
      _tc_pre_body,
      grid=(N // BR,),
      in_specs=[
          pl.BlockSpec((NCORE, BR, 1), _b3spec),
          pl.BlockSpec((BR, x.shape[1]), lambda i: (i, 0)),
          pl.BlockSpec(w.shape, lambda i: (0, 0)),
      ],
      out_specs=[
          pl.BlockSpec((BR, 1), lambda i: (i, 0)),
          pl.BlockSpec((NCORE, BR, 64), _b3spec),
      ],
      out_shape=[
          jax.ShapeDtypeStruct((N, 1), jnp.float32),
          jax.ShapeDtypeStruct((NCORE, N, 64), jnp.float32),
      ],
  )(degp, x, w)


def _tc_mid_body(agg_ref, hp_ref, dinv_ref, b_ref, w_ref, out_ref):
  dinv = dinv_ref[...]
  t = _cat(agg_ref, hp_ref) * dinv + b_ref[...]
  t = jnp.maximum(t, 0.0)
  o = jnp.dot(t, w_ref[...], preferred_element_type=jnp.float32) * dinv
  out_ref[...] = _split(o)


def _tc_mid(agg, hp, dinv, b, w):
  return pl.pallas_call(
      _tc_mid_body,
      grid=(N // BR,),
      in_specs=[
          pl.BlockSpec((NCORE, BR, 64), _b3spec),
          pl.BlockSpec((NCORE, BR, 64), _b3spec),
          pl.BlockSpec((BR, 1), lambda i: (i, 0)),
          pl.BlockSpec((1, 128), lambda i: (0, 0)),
          pl.BlockSpec((128, 128), lambda i: (0, 0)),
      ],
      out_specs=pl.BlockSpec((NCORE, BR, 64), _b3spec),
      out_shape=jax.ShapeDtypeStruct((NCORE, N, 64), jnp.float32),
  )(agg, hp, dinv, b, w)


def _tc_act_body(agg_ref, hp_ref, dinv_ref, b_ref, out_ref):
  dinv = dinv_ref[...]
  t = _cat(agg_ref, hp_ref) * dinv + b_ref[...]
  t = jnp.maximum(t, 0.0) * dinv
  out_ref[...] = _split(t)


def _tc_act(agg, hp, dinv, b):
  return pl.pallas_call(
      _tc_act_body,
      grid=(N // BR,),
      in_specs=[
          pl.BlockSpec((NCORE, BR, 64), _b3spec),
          pl.BlockSpec((NCORE, BR, 64), _b3spec),
          pl.BlockSpec((BR, 1), lambda i: (i, 0)),
          pl.BlockSpec((1, 128), lambda i: (0, 0)),
      ],
      out_specs=pl.BlockSpec((NCORE, BR, 64), _b3spec),
      out_shape=jax.ShapeDtypeStruct((NCORE, N, 64), jnp.float32),
  )(agg, hp, dinv, b)


def _tc_fin_body(agg_ref, hp_ref, dinv_ref, w_ref, b_ref, out_ref):
  t = _cat(agg_ref, hp_ref) * dinv_ref[...]
  o = jnp.dot(t, w_ref[...], preferred_element_type=jnp.float32) + b_ref[...]
  out_ref[...] = jax.nn.sigmoid(o)


def _tc_fin(agg, hp, dinv, w, b):
  return pl.pallas_call(
      _tc_fin_body,
      grid=(N // BR,),
      in_specs=[
          pl.BlockSpec((NCORE, BR, 64), _b3spec),
          pl.BlockSpec((NCORE, BR, 64), _b3spec),
          pl.BlockSpec((BR, 1), lambda i: (i, 0)),
          pl.BlockSpec((128, 1), lambda i: (0, 0)),
          pl.BlockSpec((1, 1), lambda i: (0, 0)),
      ],
      out_specs=pl.BlockSpec((BR, 1), lambda i: (i, 0)),
      out_shape=jax.ShapeDtypeStruct((N, 1), jnp.float32),
  )(agg, hp, dinv, w, b)


def _pad_edges(src, dst):
  pad_i = jnp.arange(EPAD - src.shape[0], dtype=jnp.int32)
  srcp = jnp.concatenate([src, pad_i % N])
  dstp = jnp.concatenate([dst, N + pad_i % (NR - N)])
  return srcp, dstp


@jax.jit
def kernel(x, edge_index, W1, b1, W2, b2, W3, b3):
  ei = edge_index.astype(jnp.int32)
  src, dst = ei[0], ei[1]
  srcf, dstf = _pad_edges(src, dst)
  srcp16 = srcf.reshape(NSUB, NCH_W, CB)
  dstp16 = dstf.reshape(NSUB, NCH_W, CB)
  dstp32 = dstf.reshape(NW, NCH_N, CB)

  zeros8 = jnp.zeros((NR, DW), jnp.float32)
  zeros64 = jnp.zeros((NR, 64), jnp.float32)
  ones = jnp.ones((CB, DW), jnp.float32)

  agg = _make_agg()

  degp = _make_deg()(dstp32, zeros8, ones)
  dinv, h1p = _tc_pre(degp[:, :N, :1], x, W1)
  a1 = agg(h1p, srcp16, dstp16, zeros64)
  h2p = _tc_mid(a1[:, :N], h1p, dinv, b1.reshape(1, -1), W2)
  a2 = agg(h2p, srcp16, dstp16, zeros64)
  h3p = _tc_act(a2[:, :N], h2p, dinv, b2.reshape(1, -1))
  a3 = agg(h3p, srcp16, dstp16, zeros64)
  return _tc_fin(a3[:, :N], h3p, dinv, W3, b3.reshape(1, 1))


# speedup vs baseline: 25.0444x; 1.0799x over previous
import jax
import jax.numpy as jnp
from jax import lax
from jax.experimental import pallas as pl
from jax.experimental.pallas import tpu as pltpu
from jax.experimental.pallas import tpu_sc as plsc

N = 10000
NR = 10240
NSUB = 16
NCORE = 2
NW = NCORE * NSUB
RPS = NR // NSUB
CB = 128
NCH_W = 160
NCH_N = 80
EPAD = NSUB * NCH_W * CB
BR = 1000


def _mesh():
  return plsc.VectorSubcoreMesh(core_axis_name="c", subcore_axis_name="s")


DW = 8


def _deg_body(dstp, zeros, ones, out, acc, dst_v, ones_v, wb):
  c = lax.axis_index("c")
  s = lax.axis_index("s")
  w = c * NSUB + s
  base = s * RPS
  pltpu.sync_copy(zeros.at[pl.ds(base, RPS)], wb)
  pltpu.sync_copy(wb, acc.at[pl.ds(base, RPS)])
  pltpu.sync_copy(ones, ones_v)
  pltpu.sync_copy(dstp.at[w], dst_v)
  plsc.subcore_barrier()

  def body(j, carry):
    pltpu.sync_copy(ones_v, acc.at[dst_v.at[j]], add=True)
    return carry

  lax.fori_loop(0, NCH_N, body, 0)
  plsc.subcore_barrier()
  pltpu.sync_copy(acc.at[pl.ds(base, RPS)], wb)
  pltpu.sync_copy(wb, out.at[c, pl.ds(base, RPS)])


NS = 4
LA = 3
WBR = 64


def _agg_body(table, srcp, dstp, zeros, out,
              acc, src_v, dst_v, r0, r1, r2, r3, wb,
              g0, g1, g2, g3, s0, s1, s2, s3):
  c = lax.axis_index("c")
  s = lax.axis_index("s")
  base = s * RPS
  rows = (r0, r1, r2, r3)
  gsem = (g0, g1, g2, g3)
  ssem = (s0, s1, s2, s3)

  pltpu.sync_copy(zeros.at[pl.ds(base, WBR)], wb)
  for t in range(RPS // WBR):
    pltpu.sync_copy(wb, acc.at[pl.ds(base + t * WBR, WBR)])
  pltpu.sync_copy(srcp.at[s], src_v)
  pltpu.sync_copy(dstp.at[s], dst_v)
  plsc.subcore_barrier()

  tab = table.at[c]

  def g_start(k, m):
    pltpu.async_copy(tab.at[src_v.at[m]], rows[k], gsem[k])

  def g_wait(k, m):
    pltpu.make_async_copy(tab.at[src_v.at[m]], rows[k], gsem[k]).wait()

  def s_start(k, m):
    pltpu.async_copy(rows[k], acc.at[dst_v.at[m]], ssem[k], add=True)

  def s_wait(k, m):
    pltpu.make_async_copy(rows[k], acc.at[dst_v.at[m]], ssem[k]).wait()

  def turn(m, kk, first_group, last_group):
    kn = (kk + LA) % NS
    g_wait(kk, m)
    s_start(kk, m)
    if not (first_group and kk < NS - LA):
      s_wait(kn, m - (NS - LA))
    if not (last_group and kk >= NS - LA):
      g_start(kn, m + LA)

  for kk in range(LA):
    g_start(kk, kk)
  for kk in range(NS):
    turn(kk, kk, True, False)

  def body(i, carry):
    for kk in range(NS):
      turn(NS * i + kk, kk, False, False)
    return carry

  lax.fori_loop(1, NCH_W // NS - 1, body, 0)

  for kk in range(NS):
    turn(NCH_W - NS + kk, kk, False, True)
  for m in range(NCH_W - (NS - LA), NCH_W):
    s_wait(m % NS, m)

  plsc.subcore_barrier()
  for t in range(RPS // WBR):
    sl = pl.ds(base + t * WBR, WBR)
    pltpu.sync_copy(acc.at[sl], wb)
    pltpu.sync_copy(wb, out.at[c, sl])


def _make_deg():
  return pl.kernel(
      _deg_body,
      out_type=jax.ShapeDtypeStruct((NCORE, NR, DW), jnp.float32),
      mesh=_mesh(),
      compiler_params=pltpu.CompilerParams(use_tc_tiling_on_sc=False),
      scratch_types=[
          pltpu.MemorySpace.VMEM_SHARED((NR, DW), jnp.float32),
          pltpu.VMEM((NCH_N, CB), jnp.int32),
          pltpu.VMEM((CB, DW), jnp.float32),
          pltpu.VMEM((RPS, DW), jnp.float32),
      ],
  )


def _make_agg():
  return pl.kernel(
      _agg_body,
      out_type=jax.ShapeDtypeStruct((NCORE, NR, 64), jnp.float32),
      mesh=_mesh(),
      compiler_params=pltpu.CompilerParams(use_tc_tiling_on_sc=False),
      scratch_types=[
          pltpu.MemorySpace.VMEM_SHARED((NR, 64), jnp.float32),
          pltpu.VMEM((NCH_W, CB), jnp.int32),
          pltpu.VMEM((NCH_W, CB), jnp.int32),
          pltpu.VMEM((CB, 64), jnp.float32),
          pltpu.VMEM((CB, 64), jnp.float32),
          pltpu.VMEM((CB, 64), jnp.float32),
          pltpu.VMEM((CB, 64), jnp.float32),
          pltpu.VMEM((WBR, 64), jnp.float32),
      ] + [pltpu.SemaphoreType.DMA] * (2 * NS),
  )


def _split(h):
  return jnp.stack([h[:, :64], h[:, 64:]], axis=0)


def _cat(ref_a, ref_b):
  return jnp.concatenate([ref_a[0] + ref_b[0], ref_a[1] + ref_b[1]], axis=1)


def _b3spec(i):
  return (0, i, 0)


def _tc_pre_body(degp_ref, x_ref, w_ref, dinv_ref, hp_ref):
  deg = degp_ref[0] + degp_ref[1] + 1.0
  dinv = jnp.where(deg > 0, lax.rsqrt(deg), 0.0)
  dinv_ref[...] = dinv
  h = jnp.dot(x_ref[...], w_ref[...], preferred_element_type=jnp.float32)
  hp_ref[...] = _split(h * dinv)


def _tc_pre(degp, x, w):
  return pl.pallas_call(
      _tc_pre_body,
      grid=(N // BR,),
      in_specs=[
          pl.BlockSpec((NCORE, BR, 1), _b3spec),
          pl.BlockSpec((BR, x.shape[1]), lambda i: (i, 0)),
          pl.BlockSpec(w.shape, lambda i: (0, 0)),
      ],
      out_specs=[
          pl.BlockSpec((BR, 1), lambda i: (i, 0)),
          pl.BlockSpec((NCORE, BR, 64), _b3spec),
      ],
      out_shape=[
          jax.ShapeDtypeStruct((N, 1), jnp.float32),
          jax.ShapeDtypeStruct((NCORE, N, 64), jnp.float32),
      ],
  )(degp, x, w)


def _tc_mid_body(agg_ref, hp_ref, dinv_ref, b_ref, w_ref, out_ref):
  dinv = dinv_ref[...]
  t = _cat(agg_ref, hp_ref) * dinv + b_ref[...]
  t = jnp.maximum(t, 0.0)
  o = jnp.dot(t, w_ref[...], preferred_element_type=jnp.float32) * dinv
  out_ref[...] = _split(o)


def _tc_mid(agg, hp, dinv, b, w):
  return pl.pallas_call(
      _tc_mid_body,
      grid=(N // BR,),
      in_specs=[
          pl.BlockSpec((NCORE, BR, 64), _b3spec),
          pl.BlockSpec((NCORE, BR, 64), _b3spec),
          pl.BlockSpec((BR, 1), lambda i: (i, 0)),
          pl.BlockSpec((1, 128), lambda i: (0, 0)),
          pl.BlockSpec((128, 128), lambda i: (0, 0)),
      ],
      out_specs=pl.BlockSpec((NCORE, BR, 64), _b3spec),
      out_shape=jax.ShapeDtypeStruct((NCORE, N, 64), jnp.float32),
  )(agg, hp, dinv, b, w)


def _tc_act_body(agg_ref, hp_ref, dinv_ref, b_ref, out_ref):
  dinv = dinv_ref[...]
  t = _cat(agg_ref, hp_ref) * dinv + b_ref[...]
  t = jnp.maximum(t, 0.0) * dinv
  out_ref[...] = _split(t)


def _tc_act(agg, hp, dinv, b):
  return pl.pallas_call(
      _tc_act_body,
      grid=(N // BR,),
      in_specs=[
          pl.BlockSpec((NCORE, BR, 64), _b3spec),
          pl.BlockSpec((NCORE, BR, 64), _b3spec),
          pl.BlockSpec((BR, 1), lambda i: (i, 0)),
          pl.BlockSpec((1, 128), lambda i: (0, 0)),
      ],
      out_specs=pl.BlockSpec((NCORE, BR, 64), _b3spec),
      out_shape=jax.ShapeDtypeStruct((NCORE, N, 64), jnp.float32),
  )(agg, hp, dinv, b)


def _tc_fin_body(agg_ref, hp_ref, dinv_ref, w_ref, b_ref, out_ref):
  t = _cat(agg_ref, hp_ref) * dinv_ref[...]
  o = jnp.dot(t, w_ref[...], preferred_element_type=jnp.float32) + b_ref[...]
  out_ref[...] = jax.nn.sigmoid(o)


def _tc_fin(agg, hp, dinv, w, b):
  return pl.pallas_call(
      _tc_fin_body,
      grid=(N // BR,),
      in_specs=[
          pl.BlockSpec((NCORE, BR, 64), _b3spec),
          pl.BlockSpec((NCORE, BR, 64), _b3spec),
          pl.BlockSpec((BR, 1), lambda i: (i, 0)),
          pl.BlockSpec((128, 1), lambda i: (0, 0)),
          pl.BlockSpec((1, 1), lambda i: (0, 0)),
      ],
      out_specs=pl.BlockSpec((BR, 1), lambda i: (i, 0)),
      out_shape=jax.ShapeDtypeStruct((N, 1), jnp.float32),
  )(agg, hp, dinv, w, b)


def _pad_edges(src, dst):
  pad_i = jnp.arange(EPAD - src.shape[0], dtype=jnp.int32)
  srcp = jnp.concatenate([src, pad_i % N])
  dstp = jnp.concatenate([dst, N + pad_i % (NR - N)])
  return srcp, dstp


@jax.jit
def kernel(x, edge_index, W1, b1, W2, b2, W3, b3):
  ei = edge_index.astype(jnp.int32)
  src, dst = ei[0], ei[1]
  srcf, dstf = _pad_edges(src, dst)
  srcp16 = srcf.reshape(NSUB, NCH_W, CB)
  dstp16 = dstf.reshape(NSUB, NCH_W, CB)
  dstp32 = dstf.reshape(NW, NCH_N, CB)

  zeros8 = jnp.zeros((NR, DW), jnp.float32)
  zeros64 = jnp.zeros((NR, 64), jnp.float32)
  ones = jnp.ones((CB, DW), jnp.float32)

  agg = _make_agg()

  degp = _make_deg()(dstp32, zeros8, ones)
  dinv, h1p = _tc_pre(degp[:, :N, :1], x, W1)
  a1 = agg(h1p, srcp16, dstp16, zeros64)
  h2p = _tc_mid(a1[:, :N], h1p, dinv, b1.reshape(1, -1), W2)
  a2 = agg(h2p, srcp16, dstp16, zeros64)
  h3p = _tc_act(a2[:, :N], h2p, dinv, b2.reshape(1, -1))
  a3 = agg(h3p, srcp16, dstp16, zeros64)
  return _tc_fin(a3[:, :N], h3p, dinv, W3, b3.reshape(1, 1))


# speedup vs baseline: 26.6070x; 1.0624x over previous
import jax
import jax.numpy as jnp
from jax import lax
from jax.experimental import pallas as pl
from jax.experimental.pallas import tpu as pltpu
from jax.experimental.pallas import tpu_sc as plsc

N = 10000
NR = 10240
NSUB = 16
NCORE = 2
NW = NCORE * NSUB
RPS = NR // NSUB
CB = 128
NCH_W = 160
NCH_N = 80
EPAD = NSUB * NCH_W * CB
BR = 1000


def _mesh():
  return plsc.VectorSubcoreMesh(core_axis_name="c", subcore_axis_name="s")


DW = 8


def _deg_body(dstp, zeros, ones, out, acc, dst_v, ones_v, wb):
  c = lax.axis_index("c")
  s = lax.axis_index("s")
  w = c * NSUB + s
  base = s * RPS
  pltpu.sync_copy(zeros.at[pl.ds(base, RPS)], wb)
  pltpu.sync_copy(wb, acc.at[pl.ds(base, RPS)])
  pltpu.sync_copy(ones, ones_v)
  pltpu.sync_copy(dstp.at[w], dst_v)
  plsc.subcore_barrier()

  def body(j, carry):
    pltpu.sync_copy(ones_v, acc.at[dst_v.at[j]], add=True)
    return carry

  lax.fori_loop(0, NCH_N, body, 0)
  plsc.subcore_barrier()
  pltpu.sync_copy(acc.at[pl.ds(base, RPS)], wb)
  pltpu.sync_copy(wb, out.at[c, pl.ds(base, RPS)])


NS = 4
LA = 3
WBR = 64


def _agg_body(table, srcp, dstp, zeros, out,
              acc, src_v, dst_v, r0, r1, r2, r3, wb,
              g0, g1, g2, g3, s0, s1, s2, s3):
  c = lax.axis_index("c")
  s = lax.axis_index("s")
  base = s * RPS
  rows = (r0, r1, r2, r3)
  gsem = (g0, g1, g2, g3)
  ssem = (s0, s1, s2, s3)

  pltpu.sync_copy(zeros.at[pl.ds(base, WBR)], wb)
  for t in range(RPS // WBR):
    pltpu.sync_copy(wb, acc.at[pl.ds(base + t * WBR, WBR)])
  pltpu.sync_copy(srcp.at[s], src_v)
  pltpu.sync_copy(dstp.at[s], dst_v)
  plsc.subcore_barrier()

  tab = table.at[c]

  def g_start(k, m):
    pltpu.async_copy(tab.at[src_v.at[m]], rows[k], gsem[k])

  def g_wait(k, m):
    pltpu.make_async_copy(tab.at[src_v.at[m]], rows[k], gsem[k]).wait()

  def s_start(k, m):
    pltpu.async_copy(rows[k], acc.at[dst_v.at[m]], ssem[k], add=True)

  def s_wait(k, m):
    pltpu.make_async_copy(rows[k], acc.at[dst_v.at[m]], ssem[k]).wait()

  def turn(m, kk, first_group, last_group):
    kn = (kk + LA) % NS
    g_wait(kk, m)
    s_start(kk, m)
    if not (first_group and kk < NS - LA):
      s_wait(kn, m - (NS - LA))
    if not (last_group and kk >= NS - LA):
      g_start(kn, m + LA)

  for kk in range(LA):
    g_start(kk, kk)
  for kk in range(NS):
    turn(kk, kk, True, False)

  def body(i, carry):
    for kk in range(NS):
      turn(NS * i + kk, kk, False, False)
    return carry

  lax.fori_loop(1, NCH_W // NS - 1, body, 0)

  for kk in range(NS):
    turn(NCH_W - NS + kk, kk, False, True)
  for m in range(NCH_W - (NS - LA), NCH_W):
    s_wait(m % NS, m)

  plsc.subcore_barrier()
  for t in range(RPS // WBR):
    sl = pl.ds(base + t * WBR, WBR)
    pltpu.sync_copy(acc.at[sl], wb)
    pltpu.sync_copy(wb, out.at[c, sl])


def _make_deg():
  return pl.kernel(
      _deg_body,
      out_type=jax.ShapeDtypeStruct((NCORE, NR, DW), jnp.float32),
      mesh=_mesh(),
      compiler_params=pltpu.CompilerParams(use_tc_tiling_on_sc=False),
      scratch_types=[
          pltpu.MemorySpace.VMEM_SHARED((NR, DW), jnp.float32),
          pltpu.VMEM((NCH_N, CB), jnp.int32),
          pltpu.VMEM((CB, DW), jnp.float32),
          pltpu.VMEM((RPS, DW), jnp.float32),
      ],
  )


def _make_agg():
  return pl.kernel(
      _agg_body,
      out_type=jax.ShapeDtypeStruct((NCORE, NR, 64), jnp.float32),
      mesh=_mesh(),
      compiler_params=pltpu.CompilerParams(use_tc_tiling_on_sc=False),
      scratch_types=[
          pltpu.MemorySpace.VMEM_SHARED((NR, 64), jnp.float32),
          pltpu.VMEM((NCH_W, CB), jnp.int32),
          pltpu.VMEM((NCH_W, CB), jnp.int32),
          pltpu.VMEM((CB, 64), jnp.float32),
          pltpu.VMEM((CB, 64), jnp.float32),
          pltpu.VMEM((CB, 64), jnp.float32),
          pltpu.VMEM((CB, 64), jnp.float32),
          pltpu.VMEM((WBR, 64), jnp.float32),
      ] + [pltpu.SemaphoreType.DMA] * (2 * NS),
  )


def _split(h):
  return jnp.stack([h[:, :64], h[:, 64:]], axis=0)


def _cat(ref_a, ref_b):
  return jnp.concatenate([ref_a[0] + ref_b[0], ref_a[1] + ref_b[1]], axis=1)


def _b3spec(i):
  return (0, i, 0)


def _tc_pre_body(degp_ref, x_ref, w_ref, dinv_ref, hp_ref):
  deg = degp_ref[0] + degp_ref[1] + 1.0
  dinv = jnp.where(deg > 0, lax.rsqrt(deg), 0.0)
  dinv_ref[...] = dinv
  h = jnp.dot(x_ref[...], w_ref[...], preferred_element_type=jnp.float32)
  hp_ref[...] = _split(h * dinv)


def _tc_pre(degp, x, w):
  return pl.pallas_call(
      _tc_pre_body,
      grid=(N // BR,),
      in_specs=[
          pl.BlockSpec((NCORE, BR, 1), _b3spec),
          pl.BlockSpec((BR, x.shape[1]), lambda i: (i, 0)),
          pl.BlockSpec(w.shape, lambda i: (0, 0)),
      ],
      out_specs=[
          pl.BlockSpec((BR, 1), lambda i: (i, 0)),
          pl.BlockSpec((NCORE, BR, 64), _b3spec),
      ],
      out_shape=[
          jax.ShapeDtypeStruct((N, 1), jnp.float32),
          jax.ShapeDtypeStruct((NCORE, N, 64), jnp.float32),
      ],
  )(degp, x, w)


def _tc_mid_body(agg_ref, hp_ref, dinv_ref, b_ref, w_ref, out_ref):
  dinv = dinv_ref[...]
  t = _cat(agg_ref, hp_ref) * dinv + b_ref[...]
  t = jnp.maximum(t, 0.0)
  o = jnp.dot(t, w_ref[...], preferred_element_type=jnp.float32) * dinv
  out_ref[...] = _split(o)


def _tc_mid(agg, hp, dinv, b, w):
  return pl.pallas_call(
      _tc_mid_body,
      grid=(N // BR,),
      in_specs=[
          pl.BlockSpec((NCORE, BR, 64), _b3spec),
          pl.BlockSpec((NCORE, BR, 64), _b3spec),
          pl.BlockSpec((BR, 1), lambda i: (i, 0)),
          pl.BlockSpec((1, 128), lambda i: (0, 0)),
          pl.BlockSpec((128, 128), lambda i: (0, 0)),
      ],
      out_specs=pl.BlockSpec((NCORE, BR, 64), _b3spec),
      out_shape=jax.ShapeDtypeStruct((NCORE, N, 64), jnp.float32),
  )(agg, hp, dinv, b, w)


def _tc_act_body(agg_ref, hp_ref, dinv_ref, b_ref, out_ref):
  dinv = dinv_ref[...]
  t = _cat(agg_ref, hp_ref) * dinv + b_ref[...]
  t = jnp.maximum(t, 0.0) * dinv
  out_ref[...] = _split(t)


def _tc_act(agg, hp, dinv, b):
  return pl.pallas_call(
      _tc_act_body,
      grid=(N // BR,),
      in_specs=[
          pl.BlockSpec((NCORE, BR, 64), _b3spec),
          pl.BlockSpec((NCORE, BR, 64), _b3spec),
          pl.BlockSpec((BR, 1), lambda i: (i, 0)),
          pl.BlockSpec((1, 128), lambda i: (0, 0)),
      ],
      out_specs=pl.BlockSpec((NCORE, BR, 64), _b3spec),
      out_shape=jax.ShapeDtypeStruct((NCORE, N, 64), jnp.float32),
  )(agg, hp, dinv, b)


def _tc_fin_body(agg_ref, hp_ref, dinv_ref, w_ref, b_ref, out_ref):
  t = _cat(agg_ref, hp_ref) * dinv_ref[...]
  o = jnp.dot(t, w_ref[...], preferred_element_type=jnp.float32) + b_ref[...]
  out_ref[...] = jax.nn.sigmoid(o)


def _tc_fin(agg, hp, dinv, w, b):
  return pl.pallas_call(
      _tc_fin_body,
      grid=(N // BR,),
      in_specs=[
          pl.BlockSpec((NCORE, BR, 64), _b3spec),
          pl.BlockSpec((NCORE, BR, 64), _b3spec),
          pl.BlockSpec((BR, 1), lambda i: (i, 0)),
          pl.BlockSpec((128, 1), lambda i: (0, 0)),
          pl.BlockSpec((1, 1), lambda i: (0, 0)),
      ],
      out_specs=pl.BlockSpec((BR, 1), lambda i: (i, 0)),
      out_shape=jax.ShapeDtypeStruct((N, 1), jnp.float32),
  )(agg, hp, dinv, w, b)


def _pad_edges(src, dst):
  pad_i = jnp.arange(EPAD - src.shape[0], dtype=jnp.int32)
  srcp = jnp.concatenate([src, pad_i % N])
  dstp = jnp.concatenate([dst, N + pad_i % (NR - N)])
  return srcp, dstp


@jax.jit
def kernel(x, edge_index, W1, b1, W2, b2, W3, b3):
  ei = edge_index.astype(jnp.int32)
  src, dst = ei[0], ei[1]
  srcf, dstf = _pad_edges(src, dst)
  srcp16 = srcf.reshape(NSUB, NCH_W, CB)
  dstp16 = dstf.reshape(NSUB, NCH_W, CB)
  dstp32 = dstf.reshape(NW, NCH_N, CB)

  zeros8 = jnp.zeros((NR, DW), jnp.float32)
  zeros64 = jnp.zeros((NR, 64), jnp.float32)
  ones = jnp.ones((CB, DW), jnp.float32)

  agg = _make_agg()

  degp = _make_deg()(dstp32, zeros8, ones)
  dinv, h1p = _tc_pre(degp[..., :1], x, W1)
  a1 = agg(h1p, srcp16, dstp16, zeros64)
  h2p = _tc_mid(a1, h1p, dinv, b1.reshape(1, -1), W2)
  a2 = agg(h2p, srcp16, dstp16, zeros64)
  h3p = _tc_act(a2, h2p, dinv, b2.reshape(1, -1))
  a3 = agg(h3p, srcp16, dstp16, zeros64)
  return _tc_fin(a3, h3p, dinv, W3, b3.reshape(1, 1))


# speedup vs baseline: 28.0479x; 1.0542x over previous
import jax
import jax.numpy as jnp
from jax import lax
from jax.experimental import pallas as pl
from jax.experimental.pallas import tpu as pltpu
from jax.experimental.pallas import tpu_sc as plsc

N = 10000
NR = 10240
NSUB = 16
NCORE = 2
NW = NCORE * NSUB
RPS = NR // NSUB
CB = 128
NCH_W = 160
NCH_N = 80
EPAD = NSUB * NCH_W * CB
BR = 1000


def _mesh():
  return plsc.VectorSubcoreMesh(core_axis_name="c", subcore_axis_name="s")


DW = 8


def _deg_body(dstp, zeros, ones, out, acc, dst_v, ones_v, wb):
  c = lax.axis_index("c")
  s = lax.axis_index("s")
  w = c * NSUB + s
  base = s * RPS
  pltpu.sync_copy(zeros.at[pl.ds(base, RPS)], wb)
  pltpu.sync_copy(wb, acc.at[pl.ds(base, RPS)])
  pltpu.sync_copy(ones, ones_v)
  pltpu.sync_copy(dstp.at[w], dst_v)
  plsc.subcore_barrier()

  def body(j, carry):
    pltpu.sync_copy(ones_v, acc.at[dst_v.at[j]], add=True)
    return carry

  lax.fori_loop(0, NCH_N, body, 0)
  plsc.subcore_barrier()
  pltpu.sync_copy(acc.at[pl.ds(base, RPS)], wb)
  pltpu.sync_copy(wb, out.at[c, pl.ds(base, RPS)])


NS = 5
LA = 4
WBR = 64


def _agg_body(table, srcp, dstp, zeros, out,
              acc, src_v, dst_v, r0, r1, r2, r3, r4, wb,
              g0, g1, g2, g3, g4, s0, s1, s2, s3, s4):
  c = lax.axis_index("c")
  s = lax.axis_index("s")
  base = s * RPS
  rows = (r0, r1, r2, r3, r4)
  gsem = (g0, g1, g2, g3, g4)
  ssem = (s0, s1, s2, s3, s4)

  pltpu.sync_copy(zeros.at[pl.ds(base, WBR)], wb)
  for t in range(RPS // WBR):
    pltpu.sync_copy(wb, acc.at[pl.ds(base + t * WBR, WBR)])
  pltpu.sync_copy(srcp.at[s], src_v)
  pltpu.sync_copy(dstp.at[s], dst_v)
  plsc.subcore_barrier()

  tab = table.at[c]

  def g_start(k, m):
    pltpu.async_copy(tab.at[src_v.at[m]], rows[k], gsem[k])

  def g_wait(k, m):
    pltpu.make_async_copy(tab.at[src_v.at[m]], rows[k], gsem[k]).wait()

  def s_start(k, m):
    pltpu.async_copy(rows[k], acc.at[dst_v.at[m]], ssem[k], add=True)

  def s_wait(k, m):
    pltpu.make_async_copy(rows[k], acc.at[dst_v.at[m]], ssem[k]).wait()

  def turn(m, kk, first_group, last_group):
    kn = (kk + LA) % NS
    g_wait(kk, m)
    s_start(kk, m)
    if not (first_group and kk < NS - LA):
      s_wait(kn, m - (NS - LA))
    if not (last_group and kk >= NS - LA):
      g_start(kn, m + LA)

  for kk in range(LA):
    g_start(kk, kk)
  for kk in range(NS):
    turn(kk, kk, True, False)

  def body(i, carry):
    for kk in range(NS):
      turn(NS * i + kk, kk, False, False)
    return carry

  lax.fori_loop(1, NCH_W // NS - 1, body, 0)

  for kk in range(NS):
    turn(NCH_W - NS + kk, kk, False, True)
  for m in range(NCH_W - (NS - LA), NCH_W):
    s_wait(m % NS, m)

  plsc.subcore_barrier()
  for t in range(RPS // WBR):
    sl = pl.ds(base + t * WBR, WBR)
    pltpu.sync_copy(acc.at[sl], wb)
    pltpu.sync_copy(wb, out.at[c, sl])


def _make_deg():
  return pl.kernel(
      _deg_body,
      out_type=jax.ShapeDtypeStruct((NCORE, NR, DW), jnp.float32),
      mesh=_mesh(),
      compiler_params=pltpu.CompilerParams(use_tc_tiling_on_sc=False),
      scratch_types=[
          pltpu.MemorySpace.VMEM_SHARED((NR, DW), jnp.float32),
          pltpu.VMEM((NCH_N, CB), jnp.int32),
          pltpu.VMEM((CB, DW), jnp.float32),
          pltpu.VMEM((RPS, DW), jnp.float32),
      ],
  )


def _make_agg():
  return pl.kernel(
      _agg_body,
      out_type=jax.ShapeDtypeStruct((NCORE, NR, 64), jnp.float32),
      mesh=_mesh(),
      compiler_params=pltpu.CompilerParams(use_tc_tiling_on_sc=False),
      scratch_types=[
          pltpu.MemorySpace.VMEM_SHARED((NR, 64), jnp.float32),
          pltpu.VMEM((NCH_W, CB), jnp.int32),
          pltpu.VMEM((NCH_W, CB), jnp.int32),
          pltpu.VMEM((CB, 64), jnp.float32),
          pltpu.VMEM((CB, 64), jnp.float32),
          pltpu.VMEM((CB, 64), jnp.float32),
          pltpu.VMEM((CB, 64), jnp.float32),
          pltpu.VMEM((CB, 64), jnp.float32),
          pltpu.VMEM((WBR, 64), jnp.float32),
      ] + [pltpu.SemaphoreType.DMA] * (2 * NS),
  )


def _split(h):
  return jnp.stack([h[:, :64], h[:, 64:]], axis=0)


def _cat(ref_a, ref_b):
  return jnp.concatenate([ref_a[0] + ref_b[0], ref_a[1] + ref_b[1]], axis=1)


def _b3spec(i):
  return (0, i, 0)


def _tc_pre_body(degp_ref, x_ref, w_ref, dinv_ref, hp_ref):
  deg = degp_ref[0] + degp_ref[1] + 1.0
  dinv = jnp.where(deg > 0, lax.rsqrt(deg), 0.0)
  dinv_ref[...] = dinv
  h = jnp.dot(x_ref[...], w_ref[...], preferred_element_type=jnp.float32)
  hp_ref[...] = _split(h * dinv)


def _tc_pre(degp, x, w):
  return pl.pallas_call(
      _tc_pre_body,
      grid=(N // BR,),
      in_specs=[
          pl.BlockSpec((NCORE, BR, 1), _b3spec),
          pl.BlockSpec((BR, x.shape[1]), lambda i: (i, 0)),
          pl.BlockSpec(w.shape, lambda i: (0, 0)),
      ],
      out_specs=[
          pl.BlockSpec((BR, 1), lambda i: (i, 0)),
          pl.BlockSpec((NCORE, BR, 64), _b3spec),
      ],
      out_shape=[
          jax.ShapeDtypeStruct((N, 1), jnp.float32),
          jax.ShapeDtypeStruct((NCORE, N, 64), jnp.float32),
      ],
  )(degp, x, w)


def _tc_mid_body(agg_ref, hp_ref, dinv_ref, b_ref, w_ref, out_ref):
  dinv = dinv_ref[...]
  t = _cat(agg_ref, hp_ref) * dinv + b_ref[...]
  t = jnp.maximum(t, 0.0)
  o = jnp.dot(t, w_ref[...], preferred_element_type=jnp.float32) * dinv
  out_ref[...] = _split(o)


def _tc_mid(agg, hp, dinv, b, w):
  return pl.pallas_call(
      _tc_mid_body,
      grid=(N // BR,),
      in_specs=[
          pl.BlockSpec((NCORE, BR, 64), _b3spec),
          pl.BlockSpec((NCORE, BR, 64), _b3spec),
          pl.BlockSpec((BR, 1), lambda i: (i, 0)),
          pl.BlockSpec((1, 128), lambda i: (0, 0)),
          pl.BlockSpec((128, 128), lambda i: (0, 0)),
      ],
      out_specs=pl.BlockSpec((NCORE, BR, 64), _b3spec),
      out_shape=jax.ShapeDtypeStruct((NCORE, N, 64), jnp.float32),
  )(agg, hp, dinv, b, w)


def _tc_act_body(agg_ref, hp_ref, dinv_ref, b_ref, out_ref):
  dinv = dinv_ref[...]
  t = _cat(agg_ref, hp_ref) * dinv + b_ref[...]
  t = jnp.maximum(t, 0.0) * dinv
  out_ref[...] = _split(t)


def _tc_act(agg, hp, dinv, b):
  return pl.pallas_call(
      _tc_act_body,
      grid=(N // BR,),
      in_specs=[
          pl.BlockSpec((NCORE, BR, 64), _b3spec),
          pl.BlockSpec((NCORE, BR, 64), _b3spec),
          pl.BlockSpec((BR, 1), lambda i: (i, 0)),
          pl.BlockSpec((1, 128), lambda i: (0, 0)),
      ],
      out_specs=pl.BlockSpec((NCORE, BR, 64), _b3spec),
      out_shape=jax.ShapeDtypeStruct((NCORE, N, 64), jnp.float32),
  )(agg, hp, dinv, b)


def _tc_fin_body(agg_ref, hp_ref, dinv_ref, w_ref, b_ref, out_ref):
  t = _cat(agg_ref, hp_ref) * dinv_ref[...]
  o = jnp.dot(t, w_ref[...], preferred_element_type=jnp.float32) + b_ref[...]
  out_ref[...] = jax.nn.sigmoid(o)


def _tc_fin(agg, hp, dinv, w, b):
  return pl.pallas_call(
      _tc_fin_body,
      grid=(N // BR,),
      in_specs=[
          pl.BlockSpec((NCORE, BR, 64), _b3spec),
          pl.BlockSpec((NCORE, BR, 64), _b3spec),
          pl.BlockSpec((BR, 1), lambda i: (i, 0)),
          pl.BlockSpec((128, 1), lambda i: (0, 0)),
          pl.BlockSpec((1, 1), lambda i: (0, 0)),
      ],
      out_specs=pl.BlockSpec((BR, 1), lambda i: (i, 0)),
      out_shape=jax.ShapeDtypeStruct((N, 1), jnp.float32),
  )(agg, hp, dinv, w, b)


def _pad_edges(src, dst):
  pad_i = jnp.arange(EPAD - src.shape[0], dtype=jnp.int32)
  srcp = jnp.concatenate([src, pad_i % N])
  dstp = jnp.concatenate([dst, N + pad_i % (NR - N)])
  return srcp, dstp


@jax.jit
def kernel(x, edge_index, W1, b1, W2, b2, W3, b3):
  ei = edge_index.astype(jnp.int32)
  src, dst = ei[0], ei[1]
  srcf, dstf = _pad_edges(src, dst)
  srcp16 = srcf.reshape(NSUB, NCH_W, CB)
  dstp16 = dstf.reshape(NSUB, NCH_W, CB)
  dstp32 = dstf.reshape(NW, NCH_N, CB)

  zeros8 = jnp.zeros((NR, DW), jnp.float32)
  zeros64 = jnp.zeros((NR, 64), jnp.float32)
  ones = jnp.ones((CB, DW), jnp.float32)

  agg = _make_agg()

  degp = _make_deg()(dstp32, zeros8, ones)
  dinv, h1p = _tc_pre(degp[..., :1], x, W1)
  a1 = agg(h1p, srcp16, dstp16, zeros64)
  h2p = _tc_mid(a1, h1p, dinv, b1.reshape(1, -1), W2)
  a2 = agg(h2p, srcp16, dstp16, zeros64)
  h3p = _tc_act(a2, h2p, dinv, b2.reshape(1, -1))
  a3 = agg(h3p, srcp16, dstp16, zeros64)
  return _tc_fin(a3, h3p, dinv, W3, b3.reshape(1, 1))
